# baseline (speedup 1.0000x reference)
import jax
import jax.numpy as jnp
from jax import lax
from jax.experimental import pallas as pl
from jax.experimental.pallas import tpu as pltpu
from jax.experimental.pallas import tpu_sc as plsc

CAP = 524288
NCH = 16
BB = 16384
NC, NS, L = 2, 16, 16
NW = NC * NS
CHUNK = BB // NW
ROWS = CAP // NW
RINV = 1.0 / float(NCH + 1)

_mesh = plsc.VectorSubcoreMesh(
    core_axis_name="c", subcore_axis_name="s", num_cores=NC, num_subcores=NS)


def _shift_down(x, iota, d):
    ind = jnp.maximum(iota - d, 0)
    return lax.gather(
        x, ind[:, None],
        lax.GatherDimensionNumbers(
            offset_dims=(), collapsed_slice_dims=(0,), start_index_map=(0,)),
        (1,), mode=lax.GatherScatterMode.PROMISE_IN_BOUNDS)


def _stage_a(maskw_h, ske_h, sp_h,
             s0_h, tw_h, tn_h,
             maskw_v, ske_v, sp_v, s0_v, tw_v, tn_v):
    wid = lax.axis_index("s") * NC + lax.axis_index("c")
    base = wid * CHUNK
    pltpu.sync_copy(maskw_h.at[pl.ds(base * 4, CHUNK * 4)], maskw_v)
    pltpu.sync_copy(ske_h, ske_v.at[pl.ds(0, BB)])
    pltpu.sync_copy(sp_h, sp_v)

    iota = lax.broadcasted_iota(jnp.int32, (L,), 0)
    ske_v[pl.ds(BB, L)] = jnp.full((L,), 2 ** 30, jnp.int32)

    for g in range(CHUNK // L):
        acc = jnp.zeros((L,), jnp.int32)
        for c in range(4):
            w = plsc.load_gather(maskw_v, [iota * 4 + (g * L * 4 + c)])
            acc = acc + lax.shift_right_logical(w * 0x01010101, 24)
        s0_v[pl.ds(g * L, L)] = (acc.astype(jnp.float32) + 1.0) * RINV
    pltpu.sync_copy(s0_v, s0_h.at[pl.ds(base, CHUNK)])

    lo = wid * ROWS
    hi = lo + ROWS

    def lower_bound(target):
        def bs(_, c):
            l, h = c
            mid = lax.div(l + h, 2)
            vkey = lax.reduce_min(ske_v[pl.ds(mid, L)], (0,))
            big = vkey < target
            return (jnp.where(big, mid + 1, l), jnp.where(big, h, mid))
        l, _ = lax.fori_loop(0, 14, bs, (jnp.int32(0), jnp.int32(BB)))
        return l

    pos_lo = lower_bound(lo)
    pos_hi = lower_bound(hi)
    g0 = lax.shift_right_logical(pos_lo, 4)
    g1 = lax.shift_right_logical(pos_hi + (L - 1), 4)

    def body(g, carry):
        ck, cm = carry
        off = g * L
        sk = ske_v[pl.ds(off, L)]
        nx = ske_v[pl.ds(off + 1, L)]
        sp = sp_v[pl.ds(off, L)]
        j = jnp.bitwise_and(sp, BB - 1)
        nn = lax.shift_right_logical(sp, 14)
        keep = jnp.logical_and(sk != nx,
                               jnp.logical_and(sk >= lo, sk < hi))
        lidx = jnp.where(keep, sk - lo, 0)
        y = j
        for d in (1, 2, 4, 8):
            shk = _shift_down(sk, iota, d)
            shy = _shift_down(y, iota, d)
            ok = jnp.logical_and(iota >= d, shk == sk)
            y = jnp.where(ok, jnp.maximum(y, shy), y)
        y = jnp.where(sk == ck, jnp.maximum(y, cm), y)
        plsc.store_scatter(tn_v, [lidx], nn, mask=keep)
        plsc.store_scatter(tw_v, [lidx], y, mask=keep)
        ck2 = lax.reduce_max(sk, (0,))
        cm2 = lax.reduce_max(jnp.where(iota == (L - 1), y, -1), (0,))
        return (ck2, cm2)

    lax.fori_loop(g0, g1, body, (jnp.int32(-1), jnp.int32(0)))
    pltpu.sync_copy(tw_v, tw_h.at[pl.ds(lo, ROWS)])
    pltpu.sync_copy(tn_v, tn_h.at[pl.ds(lo, ROWS)])


def _stage_b(idx2_h, tw_h, tn_h, s0_h, may_h, olp_h, val_h,
             out_h,
             iv_v, wv_v, nn_v, s0_v, g0_v, g1_v, g2_v, ov_v, sem):
    wid = lax.axis_index("s") * NC + lax.axis_index("c")
    r0 = wid * 4
    pltpu.sync_copy(idx2_h.at[pl.ds(r0, 4)], iv_v)
    cps = [pltpu.async_copy(tw_h.at[iv_v.at[k]], wv_v.at[k], sem)
           for k in range(4)]
    cps += [pltpu.async_copy(tn_h.at[iv_v.at[k]], nn_v.at[k], sem)
            for k in range(4)]
    for c in cps:
        c.wait()
    cps = []
    for k in range(4):
        cps.append(pltpu.async_copy(s0_h.at[wv_v.at[k]], s0_v.at[k], sem))
        cps.append(pltpu.async_copy(may_h.at[wv_v.at[k]], g0_v.at[k], sem))
        cps.append(pltpu.async_copy(olp_h.at[wv_v.at[k]], g1_v.at[k], sem))
        cps.append(pltpu.async_copy(val_h.at[wv_v.at[k]], g2_v.at[k], sem))
    for c in cps:
        c.wait()
    iota = lax.broadcasted_iota(jnp.int32, (L,), 0)
    for k in range(4):
        for m in range(128 // L):
            sl = (k, pl.ds(m * L, L))
            scale = s0_v[sl] - nn_v[sl].astype(jnp.float32) * RINV
            ebase = k * 384 + m * 48
            plsc.store_scatter(ov_v, [iota * 3 + ebase], g0_v[sl] * scale)
            plsc.store_scatter(ov_v, [iota * 3 + (ebase + 1)], g1_v[sl] * scale)
            plsc.store_scatter(ov_v, [iota * 3 + (ebase + 2)], g2_v[sl] * scale)
    pltpu.sync_copy(ov_v, out_h.at[pl.ds(wid * (CHUNK * 3), CHUNK * 3)])


_f32 = jnp.float32
_i32 = jnp.int32

_stage_a_call = pl.kernel(
    _stage_a,
    out_type=[jax.ShapeDtypeStruct((BB,), _f32),
              jax.ShapeDtypeStruct((CAP,), _i32),
              jax.ShapeDtypeStruct((CAP,), _i32)],
    mesh=_mesh,
    compiler_params=pltpu.CompilerParams(needs_layout_passes=False),
    scratch_types=[
        pltpu.VMEM((CHUNK * 4,), _i32),
        pltpu.VMEM((BB + 16,), _i32),
        pltpu.VMEM((BB,), _i32),
        pltpu.VMEM((CHUNK,), _f32),
        pltpu.VMEM((ROWS,), _i32),
        pltpu.VMEM((ROWS,), _i32),
    ],
)

_stage_b_call = pl.kernel(
    _stage_b,
    out_type=[jax.ShapeDtypeStruct((BB * 3,), _f32)],
    mesh=_mesh,
    compiler_params=pltpu.CompilerParams(needs_layout_passes=False),
    scratch_types=[
        pltpu.VMEM((4, 128), _i32),
        pltpu.VMEM((4, 128), _i32),
        pltpu.VMEM((4, 128), _i32),
        pltpu.VMEM((4, 128), _f32),
        pltpu.VMEM((4, 128), _f32),
        pltpu.VMEM((4, 128), _f32),
        pltpu.VMEM((4, 128), _f32),
        pltpu.VMEM((CHUNK * 3,), _f32),
        pltpu.SemaphoreType.DMA,
    ],
)


@jax.jit
def kernel(buf_may, buf_olp, buf_val, may_selected, old_log_prob, value,
           idx, trace_kind_id, buf_trace,
           decision_option_idx, decision_target_idx, decision_mask,
           uses_none_head, selected_indices,
           dec_opt_buf, dec_tgt_buf, dec_mask_buf, dec_none_buf, dec_sel_buf):
    mask_words = lax.bitcast_convert_type(
        decision_mask.view(jnp.uint8).reshape(BB * 4, 4), _i32)
    idx2 = idx.reshape(BB // 128, 128)
    payload = (uses_none_head.astype(_i32) << 14) | lax.iota(_i32, BB)
    ske, sp = lax.sort((idx, payload), dimension=0, is_stable=False,
                       num_keys=1)
    s0, tw, tn = _stage_a_call(mask_words, ske, sp)
    (out,) = _stage_b_call(idx2, tw, tn, s0,
                           may_selected, old_log_prob, value)
    return out.reshape(BB, 3)


# speedup vs baseline: 1.3633x; 1.3633x over previous
import jax
import jax.numpy as jnp
from jax import lax
from jax.experimental import pallas as pl
from jax.experimental.pallas import tpu as pltpu
from jax.experimental.pallas import tpu_sc as plsc

CAP = 524288
NCH = 16
BB = 16384
NC, NS, L = 2, 16, 16
NW = NC * NS
CHUNK = BB // NW
ROWS = CAP // NW
RINV = 1.0 / float(NCH + 1)

_mesh = plsc.VectorSubcoreMesh(
    core_axis_name="c", subcore_axis_name="s", num_cores=NC, num_subcores=NS)


def _shift_down(x, iota, d):
    ind = jnp.maximum(iota - d, 0)
    return lax.gather(
        x, ind[:, None],
        lax.GatherDimensionNumbers(
            offset_dims=(), collapsed_slice_dims=(0,), start_index_map=(0,)),
        (1,), mode=lax.GatherScatterMode.PROMISE_IN_BOUNDS)


def _stage_a(mask_h, ske_h, sp_h,
             s0_h, tw_h, tn_h,
             mask_v, ske_v, sp_v, s0_v, tw_v, tn_v):
    wid = lax.axis_index("s") * NC + lax.axis_index("c")
    base = wid * CHUNK
    pltpu.sync_copy(mask_h.at[pl.ds(base * NCH, CHUNK * NCH)], mask_v)
    pltpu.sync_copy(ske_h, ske_v.at[pl.ds(0, BB)])
    pltpu.sync_copy(sp_h, sp_v)

    iota = lax.broadcasted_iota(jnp.int32, (L,), 0)
    ske_v[pl.ds(BB, L)] = jnp.full((L,), 2 ** 30, jnp.int32)

    for g in range(CHUNK // L):
        acc = plsc.load_gather(mask_v, [iota * NCH + (g * L * NCH)])
        for c in range(1, NCH):
            acc = acc + plsc.load_gather(mask_v, [iota * NCH + (g * L * NCH + c)])
        s0_v[pl.ds(g * L, L)] = (acc.astype(jnp.float32) + 1.0) * RINV
    pltpu.sync_copy(s0_v, s0_h.at[pl.ds(base, CHUNK)])

    lo = wid * ROWS
    hi = lo + ROWS

    def lower_bound(target):
        def bs(_, c):
            l, h = c
            mid = lax.div(l + h, 2)
            vkey = lax.reduce_min(ske_v[pl.ds(mid, L)], (0,))
            big = vkey < target
            return (jnp.where(big, mid + 1, l), jnp.where(big, h, mid))
        l, _ = lax.fori_loop(0, 14, bs, (jnp.int32(0), jnp.int32(BB)))
        return l

    pos_lo = lower_bound(lo)
    pos_hi = lower_bound(hi)
    g0 = lax.shift_right_logical(pos_lo, 4)
    g1 = lax.shift_right_logical(pos_hi + (L - 1), 4)

    def body(g, carry):
        ck, cm = carry
        off = g * L
        sk = ske_v[pl.ds(off, L)]
        nx = ske_v[pl.ds(off + 1, L)]
        sp = sp_v[pl.ds(off, L)]
        j = jnp.bitwise_and(sp, BB - 1)
        nn = lax.shift_right_logical(sp, 14)
        keep = jnp.logical_and(sk != nx,
                               jnp.logical_and(sk >= lo, sk < hi))
        lidx = jnp.where(keep, sk - lo, 0)
        y = j
        for d in (1, 2, 4, 8):
            shk = _shift_down(sk, iota, d)
            shy = _shift_down(y, iota, d)
            ok = jnp.logical_and(iota >= d, shk == sk)
            y = jnp.where(ok, jnp.maximum(y, shy), y)
        y = jnp.where(sk == ck, jnp.maximum(y, cm), y)
        plsc.store_scatter(tn_v, [lidx], nn, mask=keep)
        plsc.store_scatter(tw_v, [lidx], y, mask=keep)
        ck2 = lax.reduce_max(sk, (0,))
        cm2 = lax.reduce_max(jnp.where(iota == (L - 1), y, -1), (0,))
        return (ck2, cm2)

    lax.fori_loop(g0, g1, body, (jnp.int32(-1), jnp.int32(0)))
    pltpu.sync_copy(tw_v, tw_h.at[pl.ds(lo, ROWS)])
    pltpu.sync_copy(tn_v, tn_h.at[pl.ds(lo, ROWS)])


def _stage_b(idx2_h, tw_h, tn_h, s0_h, may_h, olp_h, val_h,
             out_h,
             iv_v, wv_v, nn_v, s0_v, g0_v, g1_v, g2_v, ov_v, sem):
    wid = lax.axis_index("s") * NC + lax.axis_index("c")
    r0 = wid * 4
    pltpu.sync_copy(idx2_h.at[pl.ds(r0, 4)], iv_v)
    cps = [pltpu.async_copy(tw_h.at[iv_v.at[k]], wv_v.at[k], sem)
           for k in range(4)]
    cps += [pltpu.async_copy(tn_h.at[iv_v.at[k]], nn_v.at[k], sem)
            for k in range(4)]
    for c in cps:
        c.wait()
    cps = []
    for k in range(4):
        cps.append(pltpu.async_copy(s0_h.at[wv_v.at[k]], s0_v.at[k], sem))
        cps.append(pltpu.async_copy(may_h.at[wv_v.at[k]], g0_v.at[k], sem))
        cps.append(pltpu.async_copy(olp_h.at[wv_v.at[k]], g1_v.at[k], sem))
        cps.append(pltpu.async_copy(val_h.at[wv_v.at[k]], g2_v.at[k], sem))
    for c in cps:
        c.wait()
    iota = lax.broadcasted_iota(jnp.int32, (L,), 0)
    for k in range(4):
        for m in range(128 // L):
            sl = (k, pl.ds(m * L, L))
            scale = s0_v[sl] - nn_v[sl].astype(jnp.float32) * RINV
            ebase = k * 384 + m * 48
            plsc.store_scatter(ov_v, [iota * 3 + ebase], g0_v[sl] * scale)
            plsc.store_scatter(ov_v, [iota * 3 + (ebase + 1)], g1_v[sl] * scale)
            plsc.store_scatter(ov_v, [iota * 3 + (ebase + 2)], g2_v[sl] * scale)
    pltpu.sync_copy(ov_v, out_h.at[pl.ds(wid * (CHUNK * 3), CHUNK * 3)])


_f32 = jnp.float32
_i32 = jnp.int32

_stage_a_call = pl.kernel(
    _stage_a,
    out_type=[jax.ShapeDtypeStruct((BB,), _f32),
              jax.ShapeDtypeStruct((CAP,), _i32),
              jax.ShapeDtypeStruct((CAP,), _i32)],
    mesh=_mesh,
    compiler_params=pltpu.CompilerParams(needs_layout_passes=False),
    scratch_types=[
        pltpu.VMEM((CHUNK * NCH,), _i32),
        pltpu.VMEM((BB + 16,), _i32),
        pltpu.VMEM((BB,), _i32),
        pltpu.VMEM((CHUNK,), _f32),
        pltpu.VMEM((ROWS,), _i32),
        pltpu.VMEM((ROWS,), _i32),
    ],
)

_stage_b_call = pl.kernel(
    _stage_b,
    out_type=[jax.ShapeDtypeStruct((BB * 3,), _f32)],
    mesh=_mesh,
    compiler_params=pltpu.CompilerParams(needs_layout_passes=False),
    scratch_types=[
        pltpu.VMEM((4, 128), _i32),
        pltpu.VMEM((4, 128), _i32),
        pltpu.VMEM((4, 128), _i32),
        pltpu.VMEM((4, 128), _f32),
        pltpu.VMEM((4, 128), _f32),
        pltpu.VMEM((4, 128), _f32),
        pltpu.VMEM((4, 128), _f32),
        pltpu.VMEM((CHUNK * 3,), _f32),
        pltpu.SemaphoreType.DMA,
    ],
)


@jax.jit
def kernel(buf_may, buf_olp, buf_val, may_selected, old_log_prob, value,
           idx, trace_kind_id, buf_trace,
           decision_option_idx, decision_target_idx, decision_mask,
           uses_none_head, selected_indices,
           dec_opt_buf, dec_tgt_buf, dec_mask_buf, dec_none_buf, dec_sel_buf):
    mask_flat = decision_mask.astype(_i32).reshape(BB * NCH)
    idx2 = idx.reshape(BB // 128, 128)
    payload = (uses_none_head.astype(_i32) << 14) | lax.iota(_i32, BB)
    ske, sp = lax.sort((idx, payload), dimension=0, is_stable=False,
                       num_keys=1)
    s0, tw, tn = _stage_a_call(mask_flat, ske, sp)
    (out,) = _stage_b_call(idx2, tw, tn, s0,
                           may_selected, old_log_prob, value)
    return out.reshape(BB, 3)


# speedup vs baseline: 1.6896x; 1.2393x over previous
import jax
import jax.numpy as jnp
from jax import lax
from jax.experimental import pallas as pl
from jax.experimental.pallas import tpu as pltpu
from jax.experimental.pallas import tpu_sc as plsc

CAP = 524288
NCH = 16
BB = 16384
NC, NS, L = 2, 16, 16
NW = NC * NS
CHUNK = BB // NW
ROWS = CAP // NW
RINV = 1.0 / float(NCH + 1)

_mesh = plsc.VectorSubcoreMesh(
    core_axis_name="c", subcore_axis_name="s", num_cores=NC, num_subcores=NS)


def _shift_down(x, iota, d):
    ind = jnp.maximum(iota - d, 0)
    return lax.gather(
        x, ind[:, None],
        lax.GatherDimensionNumbers(
            offset_dims=(), collapsed_slice_dims=(0,), start_index_map=(0,)),
        (1,), mode=lax.GatherScatterMode.PROMISE_IN_BOUNDS)


def _stage_a(mask_h, ske_h, sp_h,
             s0_h, tw_h, tn_h,
             mask_v, ske_v, sp_v, s0_v, tw_v, tn_v):
    wid = lax.axis_index("s") * NC + lax.axis_index("c")
    base = wid * CHUNK
    pltpu.sync_copy(mask_h.at[pl.ds(base * NCH, CHUNK * NCH)], mask_v)
    pltpu.sync_copy(ske_h, ske_v.at[pl.ds(0, BB)])
    pltpu.sync_copy(sp_h, sp_v)

    iota = lax.broadcasted_iota(jnp.int32, (L,), 0)
    ske_v[pl.ds(BB, L)] = jnp.full((L,), 2 ** 30, jnp.int32)

    for g in range(CHUNK // L):
        acc = plsc.load_gather(mask_v, [iota * NCH + (g * L * NCH)])
        for c in range(1, NCH):
            acc = acc + plsc.load_gather(mask_v, [iota * NCH + (g * L * NCH + c)])
        s0_v[pl.ds(g * L, L)] = (acc.astype(jnp.float32) + 1.0) * RINV
    pltpu.sync_copy(s0_v, s0_h.at[pl.ds(base, CHUNK)])

    lo = wid * ROWS
    hi = lo + ROWS

    def lower_bound(target):
        def bs(_, c):
            l, h = c
            mid = lax.div(l + h, 2)
            vkey = lax.reduce_min(ske_v[pl.ds(mid, L)], (0,))
            big = vkey < target
            return (jnp.where(big, mid + 1, l), jnp.where(big, h, mid))
        l, _ = lax.fori_loop(0, 14, bs, (jnp.int32(0), jnp.int32(BB)))
        return l

    pos_lo = lower_bound(lo)
    pos_hi = lower_bound(hi)
    g0 = lax.shift_right_logical(pos_lo, 4)
    g1 = lax.shift_right_logical(pos_hi + (L - 1), 4)

    def body(g, carry):
        ck, cm = carry
        off = g * L
        sk = ske_v[pl.ds(off, L)]
        nx = ske_v[pl.ds(off + 1, L)]
        sp = sp_v[pl.ds(off, L)]
        j = jnp.bitwise_and(sp, BB - 1)
        nn = lax.shift_right_logical(sp, 14)
        keep = jnp.logical_and(sk != nx,
                               jnp.logical_and(sk >= lo, sk < hi))
        lidx = jnp.where(keep, sk - lo, 0)
        y = j
        for d in (1, 2, 4, 8):
            shk = _shift_down(sk, iota, d)
            shy = _shift_down(y, iota, d)
            ok = jnp.logical_and(iota >= d, shk == sk)
            y = jnp.where(ok, jnp.maximum(y, shy), y)
        y = jnp.where(sk == ck, jnp.maximum(y, cm), y)
        plsc.store_scatter(tn_v, [lidx], nn, mask=keep)
        plsc.store_scatter(tw_v, [lidx], y, mask=keep)
        ck2 = lax.reduce_max(sk, (0,))
        cm2 = lax.reduce_max(jnp.where(iota == (L - 1), y, -1), (0,))
        return (ck2, cm2)

    lax.fori_loop(g0, g1, body, (jnp.int32(-1), jnp.int32(0)))
    pltpu.sync_copy(tw_v, tw_h.at[pl.ds(lo, ROWS)])
    pltpu.sync_copy(tn_v, tn_h.at[pl.ds(lo, ROWS)])


def _stage_b(idx2_h, tw_h, tn_h, s0_h, may_h, olp_h, val_h,
             o0_h, o1_h, o2_h,
             iv_v, wv_v, nn_v, s0_v, g0_v, g1_v, g2_v, sem):
    wid = lax.axis_index("s") * NC + lax.axis_index("c")
    r0 = wid * 4
    pltpu.sync_copy(idx2_h.at[pl.ds(r0, 4)], iv_v)
    cps = [pltpu.async_copy(tw_h.at[iv_v.at[k]], wv_v.at[k], sem)
           for k in range(4)]
    cps += [pltpu.async_copy(tn_h.at[iv_v.at[k]], nn_v.at[k], sem)
            for k in range(4)]
    for c in cps:
        c.wait()
    cps = []
    for k in range(4):
        cps.append(pltpu.async_copy(s0_h.at[wv_v.at[k]], s0_v.at[k], sem))
        cps.append(pltpu.async_copy(may_h.at[wv_v.at[k]], g0_v.at[k], sem))
        cps.append(pltpu.async_copy(olp_h.at[wv_v.at[k]], g1_v.at[k], sem))
        cps.append(pltpu.async_copy(val_h.at[wv_v.at[k]], g2_v.at[k], sem))
    for c in cps:
        c.wait()
    for k in range(4):
        for m in range(128 // L):
            sl = (k, pl.ds(m * L, L))
            scale = s0_v[sl] - nn_v[sl].astype(jnp.float32) * RINV
            g0_v[sl] = g0_v[sl] * scale
            g1_v[sl] = g1_v[sl] * scale
            g2_v[sl] = g2_v[sl] * scale
    pltpu.sync_copy(g0_v, o0_h.at[pl.ds(r0, 4)])
    pltpu.sync_copy(g1_v, o1_h.at[pl.ds(r0, 4)])
    pltpu.sync_copy(g2_v, o2_h.at[pl.ds(r0, 4)])


_f32 = jnp.float32
_i32 = jnp.int32

_stage_a_call = pl.kernel(
    _stage_a,
    out_type=[jax.ShapeDtypeStruct((BB,), _f32),
              jax.ShapeDtypeStruct((CAP,), _i32),
              jax.ShapeDtypeStruct((CAP,), _i32)],
    mesh=_mesh,
    compiler_params=pltpu.CompilerParams(needs_layout_passes=False),
    scratch_types=[
        pltpu.VMEM((CHUNK * NCH,), _i32),
        pltpu.VMEM((BB + 16,), _i32),
        pltpu.VMEM((BB,), _i32),
        pltpu.VMEM((CHUNK,), _f32),
        pltpu.VMEM((ROWS,), _i32),
        pltpu.VMEM((ROWS,), _i32),
    ],
)

_stage_b_call = pl.kernel(
    _stage_b,
    out_type=[jax.ShapeDtypeStruct((BB // 128, 128), _f32)] * 3,
    mesh=_mesh,
    compiler_params=pltpu.CompilerParams(needs_layout_passes=False),
    scratch_types=[
        pltpu.VMEM((4, 128), _i32),
        pltpu.VMEM((4, 128), _i32),
        pltpu.VMEM((4, 128), _i32),
        pltpu.VMEM((4, 128), _f32),
        pltpu.VMEM((4, 128), _f32),
        pltpu.VMEM((4, 128), _f32),
        pltpu.VMEM((4, 128), _f32),
        pltpu.SemaphoreType.DMA,
    ],
)


@jax.jit
def kernel(buf_may, buf_olp, buf_val, may_selected, old_log_prob, value,
           idx, trace_kind_id, buf_trace,
           decision_option_idx, decision_target_idx, decision_mask,
           uses_none_head, selected_indices,
           dec_opt_buf, dec_tgt_buf, dec_mask_buf, dec_none_buf, dec_sel_buf):
    mask_flat = decision_mask.astype(_i32).reshape(BB * NCH)
    idx2 = idx.reshape(BB // 128, 128)
    payload = (uses_none_head.astype(_i32) << 14) | lax.iota(_i32, BB)
    ske, sp = lax.sort((idx, payload), dimension=0, is_stable=False,
                       num_keys=1)
    s0, tw, tn = _stage_a_call(mask_flat, ske, sp)
    o0, o1, o2 = _stage_b_call(idx2, tw, tn, s0,
                               may_selected, old_log_prob, value)
    return jnp.stack(
        [o0.reshape(BB), o1.reshape(BB), o2.reshape(BB)], axis=-1)
